# Initial kernel scaffold; baseline (speedup 1.0000x reference)
import jax
import jax.numpy as jnp
from jax.experimental import pallas as pl


def kernel(indices, table):
    raise NotImplementedError("write your pallas kernel here")


# speedup vs baseline: 1.0231x; 1.0231x over previous
import functools

import jax
import jax.numpy as jnp
from jax import lax
from jax.experimental import pallas as pl
from jax.experimental.pallas import tpu as pltpu
from jax.experimental.pallas import tpu_sc as plsc

NUM_EMBS = 1000000
EMB_SIZE = 32
BATCH = 16384
HIST = 50

_NC = 2
_NS = 16
_NW = _NC * _NS

_B = BATCH * HIST
_CH = 128
_NCHUNKS = _B // _CH
_CPW = _NCHUNKS // _NW


def _make_gather():
  mesh = plsc.VectorSubcoreMesh(core_axis_name="c", subcore_axis_name="s")

  @functools.partial(
      pl.kernel,
      mesh=mesh,
      compiler_params=pltpu.CompilerParams(use_tc_tiling_on_sc=False),
      out_type=jax.ShapeDtypeStruct((_B, EMB_SIZE), jnp.float32),
      scratch_types=[
          pltpu.VMEM((_CPW, _CH), jnp.int32),
          pltpu.VMEM((_CH, EMB_SIZE), jnp.float32),
          pltpu.SemaphoreType.DMA,
      ],
  )
  def gather_kernel(idx_hbm, tab_hbm, out_hbm, idx_v, rows_v, sem):
    wid = lax.axis_index("s") * _NC + lax.axis_index("c")
    base_chunk = wid * _CPW
    pltpu.sync_copy(idx_hbm.at[pl.ds(base_chunk, _CPW)], idx_v)

    def step(i, carry):
      pltpu.async_copy(tab_hbm.at[idx_v.at[i]], rows_v, sem).wait()
      pltpu.sync_copy(rows_v, out_hbm.at[pl.ds((base_chunk + i) * _CH, _CH)])
      return carry

    lax.fori_loop(0, _CPW, step, 0)

  return gather_kernel


_gather = _make_gather()


def kernel(indices, table):
  idx = indices.reshape(-1).astype(jnp.int32).reshape(_NCHUNKS, _CH)
  out = _gather(idx, table)
  return out.reshape(BATCH, HIST, EMB_SIZE)


# speedup vs baseline: 1.7928x; 1.7524x over previous
import functools

import jax
import jax.numpy as jnp
from jax import lax
from jax.experimental import pallas as pl
from jax.experimental.pallas import tpu as pltpu
from jax.experimental.pallas import tpu_sc as plsc

NUM_EMBS = 1000000
EMB_SIZE = 32
BATCH = 16384
HIST = 50

_NC = 2
_NS = 16
_NW = _NC * _NS

_SPW = BATCH // _NW
_G = 16
_NB = 2
_NGRP = _SPW // _G


def _make_gather():
  mesh = plsc.VectorSubcoreMesh(core_axis_name="c", subcore_axis_name="s")

  @functools.partial(
      pl.kernel,
      mesh=mesh,
      compiler_params=pltpu.CompilerParams(use_tc_tiling_on_sc=False),
      out_type=jax.ShapeDtypeStruct((BATCH, HIST, EMB_SIZE), jnp.float32),
      scratch_types=[
          pltpu.VMEM((_SPW, HIST), jnp.int32),
          pltpu.VMEM((_NB, _G, HIST, EMB_SIZE), jnp.float32),
          pltpu.SemaphoreType.DMA,
          pltpu.SemaphoreType.DMA((_NB,)),
      ],
  )
  def gather_kernel(idx_hbm, tab_hbm, out_hbm, idx_v, rows_v, gsem, wsem):
    wid = lax.axis_index("s") * _NC + lax.axis_index("c")
    base_samp = wid * _SPW
    pltpu.sync_copy(idx_hbm.at[pl.ds(base_samp, _SPW)], idx_v)

    def group_pair(gp, carry):
      for b in range(_NB):
        g = gp * _NB + b
        buf = rows_v.at[b]
        @pl.when(gp > 0)
        def _():
          pltpu.make_async_copy(
              buf, out_hbm.at[pl.ds(base_samp, _G)], wsem.at[b]).wait()

        handles = []
        for j in range(_G):
          handles.append(pltpu.async_copy(
              tab_hbm.at[idx_v.at[g * _G + j]], buf.at[j], gsem))
        for h in handles:
          h.wait()
        pltpu.async_copy(buf, out_hbm.at[pl.ds(base_samp + g * _G, _G)],
                         wsem.at[b])
      return carry

    lax.fori_loop(0, _NGRP // _NB, group_pair, 0)
    for b in range(_NB):
      pltpu.make_async_copy(
          rows_v.at[b], out_hbm.at[pl.ds(base_samp, _G)], wsem.at[b]).wait()

  return gather_kernel


_gather = _make_gather()


def kernel(indices, table):
  return _gather(indices.astype(jnp.int32), table)
